# baseline (speedup 1.0000x reference)
import functools

import jax
import jax.numpy as jnp
from jax import lax
from jax.experimental import pallas as pl
from jax.experimental.pallas import tpu as pltpu
from jax.experimental.pallas import tpu_sc as plsc

_NC = 2
_NS = 16
_NW = _NC * _NS

_W = 6400


def _build(C: int, n_in: int, n_out: int):
    assert n_out % _W == 0
    n_win = n_out // _W
    per_worker = -(-n_win // _NW)

    mesh = plsc.VectorSubcoreMesh(core_axis_name="c", subcore_axis_name="s")

    @functools.partial(
        pl.kernel,
        mesh=mesh,
        out_type=jax.ShapeDtypeStruct((C, n_out), jnp.float32),
        scratch_types=[
            pltpu.VMEM((_W,), jnp.int32),
            pltpu.VMEM((C, _W), jnp.float32),
            pltpu.SemaphoreType.DMA,
        ],
        compiler_params=pltpu.CompilerParams(
            use_tc_tiling_on_sc=False, needs_layout_passes=False
        ),
    )
    def gather_kernel(feat_hbm, idx_hbm, out_hbm, idx_v, trans_v, sem):
        wid = lax.axis_index("s") * _NC + lax.axis_index("c")

        def win(t, carry):
            w = t * _NW + wid

            @pl.when(w < n_win)
            def _():
                base = w * _W
                pltpu.sync_copy(idx_hbm.at[pl.ds(base, _W)], idx_v)
                cps = [
                    pltpu.async_copy(
                        feat_hbm.at[c].at[idx_v], trans_v.at[c], sem
                    )
                    for c in range(C)
                ]
                for cp in cps:
                    cp.wait()
                pltpu.sync_copy(trans_v, out_hbm.at[:, pl.ds(base, _W)])

            return carry

        lax.fori_loop(0, per_worker, win, 0)

    return gather_kernel


def kernel(input_features, aprs, level_deltas):
    B, C, n_in = input_features.shape
    n_out = aprs.shape[0]
    feat = input_features.reshape(B * C, n_in)
    out = _build(B * C, n_in, n_out)(feat, aprs)
    return out.reshape(B, C, n_out)


# speedup vs baseline: 1.1585x; 1.1585x over previous
import functools

import jax
import jax.numpy as jnp
from jax import lax
from jax.experimental import pallas as pl
from jax.experimental.pallas import tpu as pltpu
from jax.experimental.pallas import tpu_sc as plsc

_NC = 2
_NS = 16
_NW = _NC * _NS
_L = 16

_W = 5000
_K = 2000


def _build(C: int, n_in: int, n_out: int):
    assert n_out % _W == 0 and n_in % _K == 0
    n_win = n_out // _W
    win_per_worker = -(-n_win // _NW)
    n_chunk = n_in // _K
    chunk_per_tile = -(-n_chunk // _NS)

    mesh = plsc.VectorSubcoreMesh(core_axis_name="c", subcore_axis_name="s")

    @functools.partial(
        pl.kernel,
        mesh=mesh,
        out_type=(
            jax.ShapeDtypeStruct((C, n_out), jnp.float32),
            jax.ShapeDtypeStruct((_NC, n_in, C), jnp.float32),
        ),
        scratch_types=[
            pltpu.VMEM((_W,), jnp.int32),
            pltpu.VMEM((_W, C), jnp.float32),
            pltpu.VMEM((C, _W), jnp.float32),
            pltpu.VMEM((C, _K), jnp.float32),
            pltpu.VMEM((_K, C), jnp.float32),
            pltpu.SemaphoreType.DMA,
        ],
        compiler_params=pltpu.CompilerParams(
            use_tc_tiling_on_sc=False, needs_layout_passes=False
        ),
    )
    def gather_kernel(
        feat_hbm, idx_hbm, out_hbm, tbl_hbm,
        idx_v, rows_v, trans_v, slab_v, rowb_v, sem,
    ):
        core = lax.axis_index("c")
        sub = lax.axis_index("s")
        wid = sub * _NC + core
        lane = lax.iota(jnp.int32, _L)
        c_splat = [jnp.full((_L,), c, dtype=jnp.int32) for c in range(C)]

        def chunk(t, carry):
            k = t * _NS + sub

            @pl.when(k < n_chunk)
            def _():
                base = k * _K
                pltpu.sync_copy(feat_hbm.at[:, pl.ds(base, _K)], slab_v)

                def grp(g, c2):
                    j = g * _L
                    j_idx = j + lane
                    for c in range(C):
                        vals = slab_v[c, pl.ds(j, _L)]
                        plsc.store_scatter(rowb_v, [j_idx, c_splat[c]], vals)
                    return c2

                lax.fori_loop(0, _K // _L, grp, 0)
                pltpu.sync_copy(rowb_v, tbl_hbm.at[core].at[pl.ds(base, _K), :])

            return carry

        lax.fori_loop(0, chunk_per_tile, chunk, 0)
        plsc.subcore_barrier()

        my_tbl = tbl_hbm.at[core]

        def win(t, carry):
            w = t * _NW + wid

            @pl.when(w < n_win)
            def _():
                base = w * _W
                pltpu.sync_copy(idx_hbm.at[pl.ds(base, _W)], idx_v)
                pltpu.async_copy(my_tbl.at[idx_v], rows_v, sem).wait()

                def grp(g, c2):
                    j = g * _L
                    j_idx = j + lane
                    for c in range(C):
                        vals = plsc.load_gather(rows_v, [j_idx, c_splat[c]])
                        trans_v[c, pl.ds(j, _L)] = vals
                    return c2

                lax.fori_loop(0, _W // _L, grp, 0)
                pltpu.sync_copy(trans_v, out_hbm.at[:, pl.ds(base, _W)])

            return carry

        lax.fori_loop(0, win_per_worker, win, 0)

    return gather_kernel


def kernel(input_features, aprs, level_deltas):
    B, C, n_in = input_features.shape
    n_out = aprs.shape[0]
    feat = input_features.reshape(B * C, n_in)
    out, _ = _build(B * C, n_in, n_out)(feat, aprs)
    return out.reshape(B, C, n_out)
